# Initial kernel scaffold; baseline (speedup 1.0000x reference)
import jax
import jax.numpy as jnp
from jax.experimental import pallas as pl


def kernel(x, gate_W, gate_b, W1, b1, W2, b2):
    raise NotImplementedError("write your pallas kernel here")


# speedup vs baseline: 1.5691x; 1.5691x over previous
import functools

import jax
import jax.numpy as jnp
from jax import lax
from jax.experimental import pallas as pl
from jax.experimental.pallas import tpu as pltpu
from jax.experimental.pallas import tpu_sc as plsc

NC, NS = 2, 16
NW = NC * NS
GCHUNK = 64


def _make_sc_row_gather(V, D, Bn):
    assert Bn % (NW * GCHUNK) == 0 and D % 16 == 0
    rpt = Bn // NW
    C = rpt // GCHUNK
    mesh = plsc.VectorSubcoreMesh(core_axis_name="c", subcore_axis_name="s")

    @functools.partial(
        pl.kernel,
        mesh=mesh,
        out_type=jax.ShapeDtypeStruct((Bn, D), jnp.float32),
        scratch_types=[
            pltpu.VMEM((C, GCHUNK), jnp.int32),
            pltpu.VMEM((GCHUNK, D), jnp.float32),
            pltpu.SemaphoreType.DMA,
        ],
    )
    def gather_kernel(table_hbm, ids_hbm, out_hbm, idx_v, rows_v, sem):
        wid = lax.axis_index("s") * NC + lax.axis_index("c")
        pltpu.sync_copy(ids_hbm.at[wid], idx_v)
        base = wid * rpt
        for c in range(C):
            pltpu.async_copy(table_hbm.at[idx_v.at[c]], rows_v, sem).wait()
            pltpu.sync_copy(rows_v, out_hbm.at[pl.ds(base + c * GCHUNK, GCHUNK)])

    return gather_kernel


def _ffn_body(bexp_ref, nact_ref, x_ref, w1_ref, b1_ref, w2_ref, b2_ref, o_ref,
              *, hblk):
    b = pl.program_id(0)
    k = pl.program_id(1)

    @pl.when(b < nact_ref[0])
    def _():
        xb = x_ref[...]
        w1 = w1_ref[0]
        h = lax.dot_general(xb, w1, (((1,), (1,)), ((), ())),
                            preferred_element_type=jnp.float32)
        h = jnp.maximum(h + b1_ref[0, 0, pl.ds(k * hblk, hblk)], 0.0)
        w2 = w2_ref[0]
        y = lax.dot_general(h, w2, (((1,), (1,)), ((), ())),
                            preferred_element_type=jnp.float32)

        @pl.when(k == 0)
        def _():
            o_ref[...] = y + b2_ref[0, 0]

        @pl.when(k > 0)
        def _():
            o_ref[...] += y


def kernel(x, gate_W, gate_b, W1, b1, W2, b2):
    Bn, Sn, D = x.shape
    E, H, _ = W1.shape
    T = Bn * Sn
    BLK = 256
    HBLK = 512
    assert T % BLK == 0 and H % HBLK == 0
    NB = T // BLK + E
    P = NB * BLK
    K = H // HBLK

    x_flat = x.reshape(-1, D)

    gate_logits = x_flat @ gate_W.T + gate_b
    top1 = jnp.argmax(gate_logits, axis=-1)

    oh = (top1[:, None] == jnp.arange(E)[None, :]).astype(jnp.int32)
    ranks = jnp.take_along_axis(jnp.cumsum(oh, axis=0), top1[:, None], axis=1)[:, 0] - 1
    counts = jnp.sum(oh, axis=0)
    padded = ((counts + BLK - 1) // BLK) * BLK
    pad_end = jnp.cumsum(padded)
    total = pad_end[-1]
    pad_start = pad_end - padded
    dest = (pad_start[top1] + ranks).astype(jnp.int32)
    row_ids = jnp.zeros((P,), jnp.int32).at[dest].set(
        jnp.arange(T, dtype=jnp.int32))
    pos = jnp.arange(NB, dtype=jnp.int32) * BLK
    bexp = jnp.searchsorted(pad_end, jnp.minimum(pos, total - 1),
                            side="right").astype(jnp.int32)
    nact = (total // BLK).astype(jnp.int32).reshape(1)

    g1 = _make_sc_row_gather(T, D, P)
    x_sorted = g1(x_flat, row_ids.reshape(NW, P // NW // GCHUNK, GCHUNK))

    grid_spec = pltpu.PrefetchScalarGridSpec(
        num_scalar_prefetch=2,
        grid=(NB, K),
        in_specs=[
            pl.BlockSpec((BLK, D), lambda b, k, be, na: (b, 0)),
            pl.BlockSpec((1, HBLK, D), lambda b, k, be, na: (be[b], k, 0)),
            pl.BlockSpec((1, 1, H), lambda b, k, be, na: (be[b], 0, 0)),
            pl.BlockSpec((1, D, HBLK), lambda b, k, be, na: (be[b], 0, k)),
            pl.BlockSpec((1, 1, D), lambda b, k, be, na: (be[b], 0, 0)),
        ],
        out_specs=pl.BlockSpec((BLK, D), lambda b, k, be, na: (b, 0)),
    )
    y_sorted = pl.pallas_call(
        functools.partial(_ffn_body, hblk=HBLK),
        grid_spec=grid_spec,
        out_shape=jax.ShapeDtypeStruct((P, D), jnp.float32),
        compiler_params=pltpu.CompilerParams(
            dimension_semantics=("arbitrary", "arbitrary")),
    )(bexp, nact, x_sorted, W1, b1.reshape(E, 1, H), W2, b2.reshape(E, 1, D))

    g2 = _make_sc_row_gather(P, D, T)
    out_flat = g2(y_sorted, dest.reshape(NW, T // NW // GCHUNK, GCHUNK))
    return out_flat.reshape(Bn, Sn, D)


# speedup vs baseline: 1.5907x; 1.0137x over previous
import functools

import jax
import jax.numpy as jnp
from jax import lax
from jax.experimental import pallas as pl
from jax.experimental.pallas import tpu as pltpu
from jax.experimental.pallas import tpu_sc as plsc

NC, NS = 2, 16
NW = NC * NS
GCHUNK = 64


def _make_sc_row_gather(V, D, Bn):
    assert Bn % (NW * GCHUNK) == 0 and D % 16 == 0
    rpt = Bn // NW
    C = rpt // GCHUNK
    mesh = plsc.VectorSubcoreMesh(core_axis_name="c", subcore_axis_name="s")

    @functools.partial(
        pl.kernel,
        mesh=mesh,
        out_type=jax.ShapeDtypeStruct((Bn, D), jnp.float32),
        scratch_types=[
            pltpu.VMEM((C, GCHUNK), jnp.int32),
            pltpu.VMEM((GCHUNK, D), jnp.float32),
            pltpu.SemaphoreType.DMA,
        ],
    )
    def gather_kernel(table_hbm, ids_hbm, out_hbm, idx_v, rows_v, sem):
        wid = lax.axis_index("s") * NC + lax.axis_index("c")
        pltpu.sync_copy(ids_hbm.at[wid], idx_v)
        base = wid * rpt
        for c in range(C):
            pltpu.async_copy(table_hbm.at[idx_v.at[c]], rows_v, sem).wait()
            pltpu.sync_copy(rows_v, out_hbm.at[pl.ds(base + c * GCHUNK, GCHUNK)])

    return gather_kernel


def _ffn_body(bexp_ref, nact_ref, x_ref, w1_ref, b1_ref, w2_ref, b2_ref, o_ref,
              *, hblk):
    b = pl.program_id(0)
    k = pl.program_id(1)

    @pl.when(b < nact_ref[0])
    def _():
        xb = x_ref[...]
        w1 = w1_ref[0]
        h = lax.dot_general(xb, w1, (((1,), (1,)), ((), ())),
                            preferred_element_type=jnp.float32)
        h = jnp.maximum(h + b1_ref[0, 0, pl.ds(k * hblk, hblk)], 0.0)
        w2 = w2_ref[0]
        y = lax.dot_general(h, w2, (((1,), (1,)), ((), ())),
                            preferred_element_type=jnp.float32)

        @pl.when(k == 0)
        def _():
            o_ref[...] = y + b2_ref[0, 0]

        @pl.when(k > 0)
        def _():
            o_ref[...] += y


def kernel(x, gate_W, gate_b, W1, b1, W2, b2):
    Bn, Sn, D = x.shape
    E, H, _ = W1.shape
    T = Bn * Sn
    BLK = 256
    HBLK = 512
    assert T % BLK == 0 and H % HBLK == 0
    NB = T // BLK + E
    P = NB * BLK
    K = H // HBLK

    x_flat = x.reshape(-1, D)

    gate_logits = x_flat @ gate_W.T + gate_b
    top1 = jnp.argmax(gate_logits, axis=-1)

    CB = 128
    NCB = T // CB
    oh = (top1[:, None] == jnp.arange(E)[None, :]).astype(jnp.float32)
    oh_b = oh.reshape(NCB, CB, E)
    l_incl = jnp.tril(jnp.ones((CB, CB), jnp.float32))
    blockcum = jnp.einsum("ij,bjk->bik", l_incl, oh_b,
                          preferred_element_type=jnp.float32)
    s = jnp.sum(oh_b, axis=1)
    l_excl = jnp.tril(jnp.ones((NCB, NCB), jnp.float32), -1)
    off = l_excl @ s
    cum_incl = (blockcum + off[:, None, :]).reshape(T, E)
    ranks = (jnp.sum(cum_incl * oh, axis=-1) - 1.0).astype(jnp.int32)
    counts = jnp.sum(s, axis=0).astype(jnp.int32)
    padded = ((counts + BLK - 1) // BLK) * BLK
    pad_end = (jnp.tril(jnp.ones((E, E), jnp.float32))
               @ padded.astype(jnp.float32)).astype(jnp.int32)
    total = pad_end[-1]
    pad_start = pad_end - padded
    dest = (pad_start[top1] + ranks).astype(jnp.int32)
    row_ids = jnp.zeros((P,), jnp.int32).at[dest].set(
        jnp.arange(T, dtype=jnp.int32))
    pos = jnp.arange(NB, dtype=jnp.int32) * BLK
    bexp = jnp.searchsorted(pad_end, jnp.minimum(pos, total - 1),
                            side="right").astype(jnp.int32)
    nact = (total // BLK).astype(jnp.int32).reshape(1)

    g1 = _make_sc_row_gather(T, D, P)
    x_sorted = g1(x_flat, row_ids.reshape(NW, P // NW // GCHUNK, GCHUNK))

    grid_spec = pltpu.PrefetchScalarGridSpec(
        num_scalar_prefetch=2,
        grid=(NB, K),
        in_specs=[
            pl.BlockSpec((BLK, D), lambda b, k, be, na: (b, 0)),
            pl.BlockSpec((1, HBLK, D), lambda b, k, be, na: (be[b], k, 0)),
            pl.BlockSpec((1, 1, H), lambda b, k, be, na: (be[b], 0, 0)),
            pl.BlockSpec((1, D, HBLK), lambda b, k, be, na: (be[b], 0, k)),
            pl.BlockSpec((1, 1, D), lambda b, k, be, na: (be[b], 0, 0)),
        ],
        out_specs=pl.BlockSpec((BLK, D), lambda b, k, be, na: (b, 0)),
    )
    y_sorted = pl.pallas_call(
        functools.partial(_ffn_body, hblk=HBLK),
        grid_spec=grid_spec,
        out_shape=jax.ShapeDtypeStruct((P, D), jnp.float32),
        compiler_params=pltpu.CompilerParams(
            dimension_semantics=("arbitrary", "arbitrary")),
    )(bexp, nact, x_sorted, W1, b1.reshape(E, 1, H), W2, b2.reshape(E, 1, D))

    g2 = _make_sc_row_gather(P, D, T)
    out_flat = g2(y_sorted, dest.reshape(NW, T // NW // GCHUNK, GCHUNK))
    return out_flat.reshape(Bn, Sn, D)


# speedup vs baseline: 2.6627x; 1.6740x over previous
import functools

import jax
import jax.numpy as jnp
from jax import lax
from jax.experimental import pallas as pl
from jax.experimental.pallas import tpu as pltpu
from jax.experimental.pallas import tpu_sc as plsc

NC, NS = 2, 16
NW = NC * NS
NCHUNK = 4


def _make_sc_row_gather(V, D, Bn):
    rpt = Bn // NW
    C = NCHUNK
    CH = rpt // C
    assert Bn % (NW * C) == 0 and CH % 8 == 0 and D % 16 == 0
    assert 2 * CH * D * 4 <= 500_000, "double buffers must fit TileSpmem"
    mesh = plsc.VectorSubcoreMesh(core_axis_name="c", subcore_axis_name="s")

    @functools.partial(
        pl.kernel,
        mesh=mesh,
        out_type=jax.ShapeDtypeStruct((Bn, D), jnp.float32),
        scratch_types=[
            pltpu.VMEM((C, CH), jnp.int32),
            pltpu.VMEM((CH, D), jnp.float32),
            pltpu.VMEM((CH, D), jnp.float32),
            pltpu.SemaphoreType.DMA,
            pltpu.SemaphoreType.DMA,
        ],
    )
    def gather_kernel(table_hbm, ids_hbm, out_hbm, idx_v, rows0, rows1,
                      gsem, ssem):
        wid = lax.axis_index("s") * NC + lax.axis_index("c")
        pltpu.sync_copy(ids_hbm.at[wid], idx_v)
        base = wid * rpt
        bufs = [rows0, rows1]
        gh = [None] * C
        sh = [None] * C
        gh[0] = pltpu.async_copy(table_hbm.at[idx_v.at[0]], bufs[0], gsem)
        for c in range(C):
            if c + 1 < C:
                if c >= 1:
                    sh[c - 1].wait()
                gh[c + 1] = pltpu.async_copy(
                    table_hbm.at[idx_v.at[c + 1]], bufs[(c + 1) % 2], gsem)
            gh[c].wait()
            sh[c] = pltpu.async_copy(
                bufs[c % 2], out_hbm.at[pl.ds(base + c * CH, CH)], ssem)
        sh[C - 2].wait()
        sh[C - 1].wait()

    return gather_kernel


def _ffn_body(bexp_ref, nact_ref, x_ref, w1_ref, b1_ref, w2_ref, b2_ref, o_ref,
              *, hblk):
    b = pl.program_id(0)
    k = pl.program_id(1)
    kk = (b + k) % 2

    @pl.when(b < nact_ref[0])
    def _():
        xb = x_ref[...]
        w1 = w1_ref[0]
        h = lax.dot_general(xb, w1, (((1,), (1,)), ((), ())),
                            preferred_element_type=jnp.float32)
        h = jnp.maximum(h + b1_ref[0, 0, pl.ds(kk * hblk, hblk)], 0.0)
        w2 = w2_ref[0]
        y = lax.dot_general(h, w2, (((1,), (1,)), ((), ())),
                            preferred_element_type=jnp.float32)

        @pl.when(k == 0)
        def _():
            o_ref[...] = y + b2_ref[0, 0]

        @pl.when(k > 0)
        def _():
            o_ref[...] += y


def kernel(x, gate_W, gate_b, W1, b1, W2, b2):
    Bn, Sn, D = x.shape
    E, H, _ = W1.shape
    T = Bn * Sn
    BLK = 256
    HBLK = 2048
    assert T % BLK == 0 and H % HBLK == 0
    NB = T // BLK + E
    P = NB * BLK
    K = H // HBLK

    x_flat = x.reshape(-1, D)

    gate_logits = x_flat @ gate_W.T + gate_b
    top1 = jnp.argmax(gate_logits, axis=-1)

    CB = 128
    NCB = T // CB
    oh = (top1[:, None] == jnp.arange(E)[None, :]).astype(jnp.float32)
    oh_b = oh.reshape(NCB, CB, E)
    l_incl = jnp.tril(jnp.ones((CB, CB), jnp.float32))
    blockcum = jnp.einsum("ij,bjk->bik", l_incl, oh_b,
                          preferred_element_type=jnp.float32)
    s = jnp.sum(oh_b, axis=1)
    l_excl = jnp.tril(jnp.ones((NCB, NCB), jnp.float32), -1)
    off = l_excl @ s
    cum_incl = (blockcum + off[:, None, :]).reshape(T, E)
    ranks = (jnp.sum(cum_incl * oh, axis=-1) - 1.0).astype(jnp.int32)
    counts = jnp.sum(s, axis=0).astype(jnp.int32)
    padded = ((counts + BLK - 1) // BLK) * BLK
    pad_end = (jnp.tril(jnp.ones((E, E), jnp.float32))
               @ padded.astype(jnp.float32)).astype(jnp.int32)
    total = pad_end[-1]
    pad_start = pad_end - padded
    dest = (pad_start[top1] + ranks).astype(jnp.int32)
    row_ids = jnp.zeros((P,), jnp.int32).at[dest].set(
        jnp.arange(T, dtype=jnp.int32))
    pos = jnp.arange(NB, dtype=jnp.int32) * BLK
    bexp = jnp.searchsorted(pad_end, jnp.minimum(pos, total - 1),
                            side="right").astype(jnp.int32)
    nact = (total // BLK).astype(jnp.int32).reshape(1)

    g1 = _make_sc_row_gather(T, D, P)
    x_sorted = g1(x_flat, row_ids.reshape(NW, NCHUNK, P // NW // NCHUNK))

    grid_spec = pltpu.PrefetchScalarGridSpec(
        num_scalar_prefetch=2,
        grid=(NB, K),
        in_specs=[
            pl.BlockSpec((BLK, D), lambda b, k, be, na: (b, 0)),
            pl.BlockSpec((1, HBLK, D), lambda b, k, be, na: (be[b], (b + k) % 2, 0)),
            pl.BlockSpec((1, 1, H), lambda b, k, be, na: (be[b], 0, 0)),
            pl.BlockSpec((1, D, HBLK), lambda b, k, be, na: (be[b], 0, (b + k) % 2)),
            pl.BlockSpec((1, 1, D), lambda b, k, be, na: (be[b], 0, 0)),
        ],
        out_specs=pl.BlockSpec((BLK, D), lambda b, k, be, na: (b, 0)),
    )
    y_sorted = pl.pallas_call(
        functools.partial(_ffn_body, hblk=HBLK),
        grid_spec=grid_spec,
        out_shape=jax.ShapeDtypeStruct((P, D), jnp.float32),
        compiler_params=pltpu.CompilerParams(
            dimension_semantics=("arbitrary", "arbitrary")),
    )(bexp, nact, x_sorted, W1, b1.reshape(E, 1, H), W2, b2.reshape(E, 1, D))

    g2 = _make_sc_row_gather(P, D, T)
    out_flat = g2(y_sorted, dest.reshape(NW, NCHUNK, T // NW // NCHUNK))
    return out_flat.reshape(Bn, Sn, D)
